# Initial kernel scaffold; baseline (speedup 1.0000x reference)
import jax
import jax.numpy as jnp
from jax.experimental import pallas as pl


def kernel(x, edge_index, W, b):
    raise NotImplementedError("write your pallas kernel here")


# speedup vs baseline: 14.4547x; 14.4547x over previous
import functools

import jax
import jax.numpy as jnp
from jax import lax
from jax.experimental import pallas as pl
from jax.experimental.pallas import tpu as pltpu
from jax.experimental.pallas import tpu_sc as plsc

N = 10000
E = 320000
C = 128
NP = 10240
CHUNK = 80
DEG_ROWS_PER_TILE = (E // CHUNK) // 16
W_ROWS_PER_WORKER = (E // 16) // 32
SPMM_ROWS_PER_WORKER = (E // CHUNK) // 32
NPT = NP // 16

_MESH = plsc.VectorSubcoreMesh(
    core_axis_name="c", subcore_axis_name="s", num_cores=2, num_subcores=16
)


@functools.partial(
    pl.kernel,
    out_type=jax.ShapeDtypeStruct((32, SPMM_ROWS_PER_WORKER, CHUNK), jnp.float32),
    mesh=_MESH,
    compiler_params=pltpu.CompilerParams(needs_layout_passes=False, use_tc_tiling_on_sc=False),
    scratch_types=[
        pltpu.VMEM((SPMM_ROWS_PER_WORKER, CHUNK), jnp.int32),
        pltpu.VMEM((SPMM_ROWS_PER_WORKER, CHUNK), jnp.int32),
        pltpu.VMEM((SPMM_ROWS_PER_WORKER, CHUNK), jnp.float32),
        pltpu.VMEM((CHUNK,), jnp.float32),
        pltpu.VMEM((NPT,), jnp.float32),
        pltpu.VMEM((NP,), jnp.float32),
        pltpu.VMEM_SHARED((NP,), jnp.float32),
        pltpu.VMEM_SHARED((NP,), jnp.float32),
    ],
)
def _prepass(row3_hbm, col3_hbm, a_hbm,
             rowv80, colv80, av, onesv, degv, disv, deg_sh, dis_sh):
    c = lax.axis_index("c")
    s = lax.axis_index("s")
    wid = c * 16 + s

    for i in range(CHUNK // 16):
        onesv[pl.ds(i * 16, 16)] = jnp.full((16,), 1.0, jnp.float32)
    for i in range(NPT // 16):
        degv[pl.ds(i * 16, 16)] = jnp.zeros((16,), jnp.float32)
    pltpu.sync_copy(degv, deg_sh.at[pl.ds(s * NPT, NPT)])
    plsc.subcore_barrier()

    def deg_body(j, carry):
        pltpu.sync_copy(onesv, deg_sh.at[rowv80.at[j]], add=True)
        return carry

    for half in range(2):
        pltpu.sync_copy(row3_hbm.at[s * 2 + half], rowv80)
        lax.fori_loop(0, SPMM_ROWS_PER_WORKER, deg_body, 0)
    plsc.subcore_barrier()

    pltpu.sync_copy(deg_sh.at[pl.ds(s * NPT, NPT)], degv)
    for i in range(NPT // 16):
        d = degv[pl.ds(i * 16, 16)]
        dsafe = jnp.maximum(d, 1.0)
        sq = (dsafe + 1.0) * 0.5
        for _ in range(14):
            sq = (sq + dsafe / sq) * 0.5
        degv[pl.ds(i * 16, 16)] = jnp.where(
            d > 0.0, 1.0 / sq, jnp.zeros((16,), jnp.float32))
    pltpu.sync_copy(degv, dis_sh.at[pl.ds(s * NPT, NPT)])
    plsc.subcore_barrier()

    pltpu.sync_copy(dis_sh, disv)
    pltpu.sync_copy(row3_hbm.at[wid], rowv80)
    pltpu.sync_copy(col3_hbm.at[wid], colv80)

    def w_body(j, carry):
        for k in range(CHUNK // 16):
            dr = plsc.load_gather(disv, [rowv80[j, pl.ds(k * 16, 16)]])
            dc = plsc.load_gather(disv, [colv80[j, pl.ds(k * 16, 16)]])
            av[j, pl.ds(k * 16, 16)] = -(dr * dc) - 1.0
        return carry

    lax.fori_loop(0, SPMM_ROWS_PER_WORKER, w_body, 0)
    pltpu.sync_copy(av, a_hbm.at[wid])


@functools.partial(
    pl.kernel,
    out_type=jax.ShapeDtypeStruct((2, 16, NPT, C), jnp.float32),
    mesh=_MESH,
    compiler_params=pltpu.CompilerParams(needs_layout_passes=False, use_tc_tiling_on_sc=False),
    scratch_types=[
        pltpu.VMEM((SPMM_ROWS_PER_WORKER, CHUNK), jnp.int32),
        pltpu.VMEM((SPMM_ROWS_PER_WORKER, CHUNK), jnp.int32),
        pltpu.VMEM((SPMM_ROWS_PER_WORKER, CHUNK), jnp.float32),
        pltpu.VMEM((CHUNK, C), jnp.float32),
        pltpu.VMEM((32, C), jnp.float32),
        pltpu.VMEM_SHARED((NP, C), jnp.float32),
    ],
)
def _spmm(h_hbm, col3_hbm, row3_hbm, a3_hbm, part_hbm,
          colv, rowv, av, gbuf, zbuf, acc_sh):
    c = lax.axis_index("c")
    s = lax.axis_index("s")
    wid = c * 16 + s

    for i in range(32):
        for r in range(C // 16):
            zbuf[i, pl.ds(r * 16, 16)] = jnp.zeros((16,), jnp.float32)
    for i in range(NPT // 32):
        pltpu.sync_copy(zbuf, acc_sh.at[pl.ds(s * NPT + i * 32, 32)])
    plsc.subcore_barrier()

    pltpu.sync_copy(col3_hbm.at[wid], colv)
    pltpu.sync_copy(row3_hbm.at[wid], rowv)
    pltpu.sync_copy(a3_hbm.at[wid], av)

    def body(j, carry):
        pltpu.sync_copy(h_hbm.at[colv.at[j]], gbuf)
        for k in range(CHUNK // 16):
            avv = av[j, pl.ds(k * 16, 16)]
            for t in range(16):
                jj = k * 16 + t
                aa = avv[t]
                for r in range(C // 16):
                    gbuf[jj, pl.ds(r * 16, 16)] = gbuf[jj, pl.ds(r * 16, 16)] * aa
        pltpu.sync_copy(gbuf, acc_sh.at[rowv.at[j]], add=True)
        return carry

    lax.fori_loop(0, SPMM_ROWS_PER_WORKER, body, 0)
    plsc.subcore_barrier()

    pltpu.sync_copy(acc_sh.at[pl.ds(s * NPT, NPT)], part_hbm.at[c, s])


def _combine_body(p_ref, o_ref):
    o_ref[...] = p_ref[0] + p_ref[1]


def _final_body(x_ref, t1_ref, q_ref, w_ref, b_ref, o_ref):
    s2 = q_ref[0] + q_ref[1]
    acc = jnp.dot(x_ref[...], w_ref[0], preferred_element_type=jnp.float32)
    acc = acc + jnp.dot(t1_ref[...], w_ref[1], preferred_element_type=jnp.float32)
    acc = acc + jnp.dot(s2, w_ref[2], preferred_element_type=jnp.float32)
    o_ref[...] = acc + b_ref[...]


def kernel(x, edge_index, W, b):
    row = edge_index[0]
    col = edge_index[1]
    row3 = row.reshape(32, SPMM_ROWS_PER_WORKER, CHUNK)
    col3 = col.reshape(32, SPMM_ROWS_PER_WORKER, CHUNK)

    a3 = _prepass(row3, col3)

    p = _spmm(x, col3, row3, a3).reshape(2, NP, C)[:, :N, :]
    t1 = pl.pallas_call(
        _combine_body,
        grid=(10,),
        in_specs=[pl.BlockSpec((2, N // 10, C), lambda i: (0, i, 0))],
        out_specs=pl.BlockSpec((N // 10, C), lambda i: (i, 0)),
        out_shape=jax.ShapeDtypeStruct((N, C), jnp.float32),
    )(p)

    q = _spmm(t1, col3, row3, a3).reshape(2, NP, C)[:, :N, :]

    Wc = jnp.stack([W[0] - W[2], W[1], 2.0 * W[2]])
    b2 = b.reshape(1, C)
    out = pl.pallas_call(
        _final_body,
        grid=(10,),
        in_specs=[
            pl.BlockSpec((N // 10, C), lambda i: (i, 0)),
            pl.BlockSpec((N // 10, C), lambda i: (i, 0)),
            pl.BlockSpec((2, N // 10, C), lambda i: (0, i, 0)),
            pl.BlockSpec((3, C, C), lambda i: (0, 0, 0)),
            pl.BlockSpec((1, C), lambda i: (0, 0)),
        ],
        out_specs=pl.BlockSpec((N // 10, C), lambda i: (i, 0)),
        out_shape=jax.ShapeDtypeStruct((N, C), jnp.float32),
    )(x, t1, q, Wc, b2)
    return out


# speedup vs baseline: 22.4207x; 1.5511x over previous
import functools

import jax
import jax.numpy as jnp
from jax import lax
from jax.experimental import pallas as pl
from jax.experimental.pallas import tpu as pltpu
from jax.experimental.pallas import tpu_sc as plsc

N = 10000
E = 320000
C = 128
NP = 10240
CHUNK = 80
DEG_ROWS_PER_TILE = (E // CHUNK) // 16
W_ROWS_PER_WORKER = (E // 16) // 32
SPMM_ROWS_PER_WORKER = (E // CHUNK) // 32
NPT = NP // 16

_MESH = plsc.VectorSubcoreMesh(
    core_axis_name="c", subcore_axis_name="s", num_cores=2, num_subcores=16
)


@functools.partial(
    pl.kernel,
    out_type=jax.ShapeDtypeStruct((32, SPMM_ROWS_PER_WORKER, CHUNK), jnp.float32),
    mesh=_MESH,
    compiler_params=pltpu.CompilerParams(needs_layout_passes=False, use_tc_tiling_on_sc=False),
    scratch_types=[
        pltpu.VMEM((SPMM_ROWS_PER_WORKER, CHUNK), jnp.int32),
        pltpu.VMEM((SPMM_ROWS_PER_WORKER, CHUNK), jnp.int32),
        pltpu.VMEM((SPMM_ROWS_PER_WORKER, CHUNK), jnp.float32),
        pltpu.VMEM((CHUNK,), jnp.float32),
        pltpu.VMEM((NPT,), jnp.float32),
        pltpu.VMEM((NP,), jnp.float32),
        pltpu.VMEM_SHARED((NP,), jnp.float32),
        pltpu.VMEM_SHARED((NP,), jnp.float32),
    ],
)
def _prepass(row3_hbm, col3_hbm, a_hbm,
             rowv80, colv80, av, onesv, degv, disv, deg_sh, dis_sh):
    c = lax.axis_index("c")
    s = lax.axis_index("s")
    wid = c * 16 + s

    for i in range(CHUNK // 16):
        onesv[pl.ds(i * 16, 16)] = jnp.full((16,), 1.0, jnp.float32)
    for i in range(NPT // 16):
        degv[pl.ds(i * 16, 16)] = jnp.zeros((16,), jnp.float32)
    pltpu.sync_copy(degv, deg_sh.at[pl.ds(s * NPT, NPT)])
    plsc.subcore_barrier()

    def deg_body(j, carry):
        pltpu.sync_copy(onesv, deg_sh.at[rowv80.at[j]], add=True)
        return carry

    for half in range(2):
        pltpu.sync_copy(row3_hbm.at[s * 2 + half], rowv80)
        lax.fori_loop(0, SPMM_ROWS_PER_WORKER, deg_body, 0)
    plsc.subcore_barrier()

    pltpu.sync_copy(deg_sh.at[pl.ds(s * NPT, NPT)], degv)
    for i in range(NPT // 16):
        d = degv[pl.ds(i * 16, 16)]
        dsafe = jnp.maximum(d, 1.0)
        sq = (dsafe + 1.0) * 0.5
        for _ in range(14):
            sq = (sq + dsafe / sq) * 0.5
        degv[pl.ds(i * 16, 16)] = jnp.where(
            d > 0.0, 1.0 / sq, jnp.zeros((16,), jnp.float32))
    pltpu.sync_copy(degv, dis_sh.at[pl.ds(s * NPT, NPT)])
    plsc.subcore_barrier()

    pltpu.sync_copy(dis_sh, disv)
    pltpu.sync_copy(row3_hbm.at[wid], rowv80)
    pltpu.sync_copy(col3_hbm.at[wid], colv80)

    def w_body(j, carry):
        for k in range(CHUNK // 16):
            dr = plsc.load_gather(disv, [rowv80[j, pl.ds(k * 16, 16)]])
            dc = plsc.load_gather(disv, [colv80[j, pl.ds(k * 16, 16)]])
            av[j, pl.ds(k * 16, 16)] = -(dr * dc) - 1.0
        return carry

    lax.fori_loop(0, SPMM_ROWS_PER_WORKER, w_body, 0)
    pltpu.sync_copy(av, a_hbm.at[wid])


@functools.partial(
    pl.kernel,
    out_type=jax.ShapeDtypeStruct((2, 16, 625, C), jnp.float32),
    mesh=_MESH,
    compiler_params=pltpu.CompilerParams(needs_layout_passes=False, use_tc_tiling_on_sc=False),
    scratch_types=[
        pltpu.VMEM((SPMM_ROWS_PER_WORKER, CHUNK), jnp.int32),
        pltpu.VMEM((SPMM_ROWS_PER_WORKER, CHUNK), jnp.int32),
        pltpu.VMEM((SPMM_ROWS_PER_WORKER, CHUNK), jnp.float32),
        pltpu.VMEM((CHUNK, C), jnp.float32),
        pltpu.VMEM((CHUNK, C), jnp.float32),
        pltpu.VMEM_SHARED((N, C), jnp.float32),
        pltpu.SemaphoreType.DMA,
        pltpu.SemaphoreType.DMA,
    ],
)
def _spmm(h_hbm, col3_hbm, row3_hbm, a3_hbm, part_hbm,
          colv, rowv, av, gbuf0, gbuf1, acc_sh, sem0, sem1):
    c = lax.axis_index("c")
    s = lax.axis_index("s")
    wid = c * 16 + s

    for i in range(CHUNK):
        for r in range(C // 16):
            gbuf0[i, pl.ds(r * 16, 16)] = jnp.zeros((16,), jnp.float32)
    for i in range(7):
        pltpu.sync_copy(gbuf0, acc_sh.at[pl.ds(s * 625 + i * 80, 80)])
    pltpu.sync_copy(gbuf0.at[pl.ds(0, 65)], acc_sh.at[pl.ds(s * 625 + 560, 65)])
    plsc.subcore_barrier()

    pltpu.sync_copy(col3_hbm.at[wid], colv)
    pltpu.sync_copy(row3_hbm.at[wid], rowv)
    pltpu.sync_copy(a3_hbm.at[wid], av)

    def scale_scatter(m, gbuf):
        for k in range(CHUNK // 16):
            avv = av[m, pl.ds(k * 16, 16)]
            for t in range(16):
                jj = k * 16 + t
                aa = avv[t]
                for r in range(C // 16):
                    gbuf[jj, pl.ds(r * 16, 16)] = gbuf[jj, pl.ds(r * 16, 16)] * aa
        pltpu.sync_copy(gbuf, acc_sh.at[rowv.at[m]], add=True)

    pltpu.async_copy(h_hbm.at[colv.at[0]], gbuf0, sem0)

    def body(j, carry):
        m0 = 2 * j
        pltpu.async_copy(h_hbm.at[colv.at[m0 + 1]], gbuf1, sem1)
        pltpu.make_async_copy(h_hbm.at[colv.at[m0]], gbuf0, sem0).wait()
        scale_scatter(m0, gbuf0)
        pltpu.async_copy(h_hbm.at[colv.at[m0 + 2]], gbuf0, sem0)
        pltpu.make_async_copy(h_hbm.at[colv.at[m0 + 1]], gbuf1, sem1).wait()
        scale_scatter(m0 + 1, gbuf1)
        return carry

    lax.fori_loop(0, (SPMM_ROWS_PER_WORKER - 1) // 2, body, 0)
    last = SPMM_ROWS_PER_WORKER - 1
    pltpu.make_async_copy(h_hbm.at[colv.at[last]], gbuf0, sem0).wait()
    scale_scatter(last, gbuf0)
    plsc.subcore_barrier()

    pltpu.sync_copy(acc_sh.at[pl.ds(s * 625, 625)], part_hbm.at[c, s])


def _combine_body(p_ref, o_ref):
    o_ref[...] = p_ref[0] + p_ref[1]


def _final_body(x_ref, t1_ref, q_ref, w_ref, b_ref, o_ref):
    s2 = q_ref[0] + q_ref[1]
    acc = jnp.dot(x_ref[...], w_ref[0], preferred_element_type=jnp.float32)
    acc = acc + jnp.dot(t1_ref[...], w_ref[1], preferred_element_type=jnp.float32)
    acc = acc + jnp.dot(s2, w_ref[2], preferred_element_type=jnp.float32)
    o_ref[...] = acc + b_ref[...]


def kernel(x, edge_index, W, b):
    row = edge_index[0]
    col = edge_index[1]
    row3 = row.reshape(32, SPMM_ROWS_PER_WORKER, CHUNK)
    col3 = col.reshape(32, SPMM_ROWS_PER_WORKER, CHUNK)

    a3 = _prepass(row3, col3)

    p = _spmm(x, col3, row3, a3).reshape(2, N, C)
    t1 = pl.pallas_call(
        _combine_body,
        grid=(10,),
        in_specs=[pl.BlockSpec((2, N // 10, C), lambda i: (0, i, 0))],
        out_specs=pl.BlockSpec((N // 10, C), lambda i: (i, 0)),
        out_shape=jax.ShapeDtypeStruct((N, C), jnp.float32),
    )(p)

    q = _spmm(t1, col3, row3, a3).reshape(2, N, C)

    Wc = jnp.stack([W[0] - W[2], W[1], 2.0 * W[2]])
    b2 = b.reshape(1, C)
    out = pl.pallas_call(
        _final_body,
        grid=(10,),
        in_specs=[
            pl.BlockSpec((N // 10, C), lambda i: (i, 0)),
            pl.BlockSpec((N // 10, C), lambda i: (i, 0)),
            pl.BlockSpec((2, N // 10, C), lambda i: (0, i, 0)),
            pl.BlockSpec((3, C, C), lambda i: (0, 0, 0)),
            pl.BlockSpec((1, C), lambda i: (0, 0)),
        ],
        out_specs=pl.BlockSpec((N // 10, C), lambda i: (i, 0)),
        out_shape=jax.ShapeDtypeStruct((N, C), jnp.float32),
    )(x, t1, q, Wc, b2)
    return out
